# baseline (device time: 30878 ns/iter reference)
import jax
import jax.numpy as jnp
from jax import lax
from jax.experimental import pallas as pl
from jax.experimental.pallas import tpu as pltpu

N_DEV = 8
B = 2
S_LOC = 128
D = 512
HQ = 4
DH = 64
HD = HQ * DH
R = B * S_LOC


def kernel(x, Wq, Wk, Wv, Wo):
    pos = jnp.arange(N_DEV * S_LOC, dtype=jnp.float32).reshape(N_DEV, S_LOC)
    inv = 1.0 / (10000.0 ** (jnp.arange(0, DH, 2, dtype=jnp.float32) / DH))
    ang = pos[:, :, None] * inv[None, None, :]
    cos_t = jnp.tile(jnp.repeat(jnp.cos(ang), 2, axis=2), (1, B, HQ))
    sin_t = jnp.tile(jnp.repeat(jnp.sin(ang), 2, axis=2), (1, B, HQ))

    def body(x_ref, wq_ref, wk_ref, wv_ref, wo_ref, cos_ref, sin_ref,
             out_ref, x_buf, scores_buf, v_t, send_sems, recv_sems):
        me = lax.axis_index("i")
        partners = [me ^ 1, me ^ 3, me ^ 4]

        barrier_sem = pltpu.get_barrier_semaphore()
        for nbr in partners:
            pl.semaphore_signal(
                barrier_sem, inc=1,
                device_id=(nbr,), device_id_type=pl.DeviceIdType.MESH,
            )
        pl.semaphore_wait(barrier_sem, len(partners))

        xf = jnp.concatenate([x_ref[0], x_ref[1]], axis=0).astype(jnp.bfloat16)
        x_buf[me] = xf

        def start_flow(idx, slot, partner):
            rdma = pltpu.make_async_remote_copy(
                src_ref=x_buf.at[pl.ds(slot, 1)],
                dst_ref=x_buf.at[pl.ds(slot, 1)],
                send_sem=send_sems.at[idx],
                recv_sem=recv_sems.at[idx],
                device_id=(partner,),
                device_id_type=pl.DeviceIdType.MESH,
            )
            rdma.start()
            return rdma

        r_own_x = start_flow(0, me, partners[0])
        r_own_y = start_flow(1, me, partners[1])
        r_own_z = start_flow(2, me, partners[2])

        wq = wq_ref[...].astype(jnp.bfloat16)
        wk = wk_ref[...].astype(jnp.bfloat16)
        wv = wv_ref[...].astype(jnp.bfloat16)

        row = lax.broadcasted_iota(jnp.int32, (HD, HD), 0)
        col = lax.broadcasted_iota(jnp.int32, (HD, HD), 1)
        even = (col % 2) == 0
        r_mat = jnp.where(even & (row == col + 1), -1.0, 0.0) + jnp.where(
            (~even) & (row == col - 1), 1.0, 0.0
        )

        def rope(t, slot):
            return (
                t * cos_ref[slot]
                + jnp.dot(t, r_mat, preferred_element_type=jnp.float32)
                * sin_ref[slot]
            )

        q = jnp.dot(xf, wq, preferred_element_type=jnp.float32)
        qb = rope(q, me).astype(jnp.bfloat16)

        m_run = [[jnp.full((S_LOC, 1), -1e30, jnp.float32) for _ in range(HQ)]
                 for _ in range(B)]

        def process_chunk(slot):
            xc = x_buf[slot]
            kb = rope(
                jnp.dot(xc, wk, preferred_element_type=jnp.float32), slot
            ).astype(jnp.bfloat16)
            vb = jnp.dot(xc, wv, preferred_element_type=jnp.float32).astype(
                jnp.bfloat16
            )
            for b in range(B):
                for hh in range(HQ):
                    qh = qb[b * S_LOC:(b + 1) * S_LOC, hh * DH:(hh + 1) * DH]
                    kc = kb[b * S_LOC:(b + 1) * S_LOC, hh * DH:(hh + 1) * DH]
                    sc = lax.dot_general(
                        qh, kc, (((1,), (1,)), ((), ())),
                        preferred_element_type=jnp.float32,
                    ) * 0.125
                    scores_buf[b, hh, :, pl.ds(slot * S_LOC, S_LOC)] = sc
                    m_run[b][hh] = jnp.maximum(
                        m_run[b][hh], jnp.max(sc, axis=1, keepdims=True)
                    )
                    v_t[b, hh, pl.ds(slot * S_LOC, S_LOC), :] = vb[
                        b * S_LOC:(b + 1) * S_LOC, hh * DH:(hh + 1) * DH
                    ]

        process_chunk(me)
        r_own_x.wait_recv()
        r_fwd_y = start_flow(3, me ^ 1, partners[1])
        r_fwd_z = start_flow(4, me ^ 1, partners[2])
        process_chunk(me ^ 1)
        r_own_y.wait_recv()
        r_fwd3_z = start_flow(5, me ^ 3, partners[2])
        process_chunk(me ^ 3)
        r_fwd_y.wait_recv()
        r_fwd2_z = start_flow(6, me ^ 2, partners[2])
        process_chunk(me ^ 2)
        r_own_z.wait_recv()
        process_chunk(me ^ 4)
        r_fwd_z.wait_recv()
        process_chunk(me ^ 5)
        r_fwd3_z.wait_recv()
        process_chunk(me ^ 7)
        r_fwd2_z.wait_recv()
        process_chunk(me ^ 6)
        for r in (r_own_x, r_own_y, r_own_z, r_fwd_y, r_fwd_z, r_fwd3_z,
                  r_fwd2_z):
            r.wait_send()

        wo = wo_ref[...].astype(jnp.bfloat16)
        for b in range(B):
            ctx_heads = []
            for hh in range(HQ):
                scores = scores_buf[b, hh]
                w = jnp.exp(scores - m_run[b][hh])
                denom = jnp.sum(w, axis=1, keepdims=True)
                acc = jnp.dot(
                    w.astype(jnp.bfloat16), v_t[b, hh],
                    preferred_element_type=jnp.float32,
                )
                ctx_heads.append(acc / denom)
            ctx_b = jnp.concatenate(ctx_heads, axis=1).astype(jnp.bfloat16)
            out_ref[b] = jnp.dot(ctx_b, wo, preferred_element_type=jnp.float32)

    return pl.pallas_call(
        body,
        out_shape=jax.ShapeDtypeStruct((B, S_LOC, D), jnp.float32),
        in_specs=[pl.BlockSpec(memory_space=pltpu.VMEM)] * 7,
        out_specs=pl.BlockSpec(memory_space=pltpu.VMEM),
        scratch_shapes=[
            pltpu.VMEM((N_DEV, R, D), jnp.bfloat16),
            pltpu.VMEM((B, HQ, S_LOC, N_DEV * S_LOC), jnp.float32),
            pltpu.VMEM((B, HQ, N_DEV * S_LOC, DH), jnp.bfloat16),
            pltpu.SemaphoreType.DMA((7,)),
            pltpu.SemaphoreType.DMA((7,)),
        ],
        compiler_params=pltpu.CompilerParams(collective_id=0),
    )(x, Wq, Wk, Wv, Wo, cos_t, sin_t)


# device time: 23624 ns/iter; 1.3071x vs baseline; 1.3071x over previous
import functools

import jax
import jax.numpy as jnp
from jax import lax
from jax.experimental import pallas as pl
from jax.experimental.pallas import tpu as pltpu

N_DEV = 8
B = 2
S_LOC = 128
D = 512
HQ = 4
DH = 64
HD = HQ * DH
R = B * S_LOC


def kernel(x, Wq, Wk, Wv, Wo):
    my = lax.axis_index("i")

    pos = (my * S_LOC).astype(jnp.float32) + jnp.arange(S_LOC, dtype=jnp.float32)
    inv = 1.0 / (10000.0 ** (jnp.arange(0, DH, 2, dtype=jnp.float32) / DH))
    ang = pos[:, None] * inv[None, :]
    cos = jnp.repeat(jnp.cos(ang), 2, axis=1)
    sin = jnp.repeat(jnp.sin(ang), 2, axis=1)
    cos_full = jnp.tile(cos, (B, HQ))
    sin_full = jnp.tile(sin, (B, HQ))

    def body(x_ref, wq_ref, wk_ref, wv_ref, wo_ref, cos_ref, sin_ref,
             out_ref, kv_buf, scores_buf, v_t, send_sems, recv_sems):
        me = lax.axis_index("i")
        partners = [me ^ 1, me ^ 3, me ^ 4]

        barrier_sem = pltpu.get_barrier_semaphore()
        for nbr in partners:
            pl.semaphore_signal(
                barrier_sem, inc=1,
                device_id=(nbr,), device_id_type=pl.DeviceIdType.MESH,
            )
        pl.semaphore_wait(barrier_sem, len(partners))

        xf = jnp.concatenate([x_ref[0], x_ref[1]], axis=0).astype(jnp.bfloat16)
        wk = wk_ref[...].astype(jnp.bfloat16)
        wv = wv_ref[...].astype(jnp.bfloat16)
        k = jnp.dot(xf, wk, preferred_element_type=jnp.float32)
        v = jnp.dot(xf, wv, preferred_element_type=jnp.float32)

        row = lax.broadcasted_iota(jnp.int32, (HD, HD), 0)
        col = lax.broadcasted_iota(jnp.int32, (HD, HD), 1)
        even = (col % 2) == 0
        r_mat = jnp.where(even & (row == col + 1), -1.0, 0.0) + jnp.where(
            (~even) & (row == col - 1), 1.0, 0.0
        )
        cosf = cos_ref[...]
        sinf = sin_ref[...]
        k_rot = k * cosf + jnp.dot(k, r_mat, preferred_element_type=jnp.float32) * sinf

        kv_buf[me, 0] = k_rot.astype(jnp.bfloat16)
        kv_buf[me, 1] = v.astype(jnp.bfloat16)

        def start_flow(idx, slot, size, partner):
            rdma = pltpu.make_async_remote_copy(
                src_ref=kv_buf.at[pl.ds(slot, size)],
                dst_ref=kv_buf.at[pl.ds(slot, size)],
                send_sem=send_sems.at[idx],
                recv_sem=recv_sems.at[idx],
                device_id=(partner,),
                device_id_type=pl.DeviceIdType.MESH,
            )
            rdma.start()
            return rdma

        m_run = [[jnp.full((S_LOC, 1), -1e30, jnp.float32) for _ in range(HQ)]
                 for _ in range(B)]

        def scores_for(slot):
            for b in range(B):
                for hh in range(HQ):
                    qh = qb[b * S_LOC:(b + 1) * S_LOC, hh * DH:(hh + 1) * DH]
                    kc = kv_buf[slot, 0, b * S_LOC:(b + 1) * S_LOC,
                                hh * DH:(hh + 1) * DH]
                    sc = lax.dot_general(
                        qh, kc, (((1,), (1,)), ((), ())),
                        preferred_element_type=jnp.float32,
                    ) * 0.125
                    scores_buf[b, hh, :, pl.ds(slot * S_LOC, S_LOC)] = (
                        sc.astype(jnp.bfloat16)
                    )
                    m_run[b][hh] = jnp.maximum(
                        m_run[b][hh], jnp.max(sc, axis=1, keepdims=True)
                    )
                    v_t[b, hh, pl.ds(slot * S_LOC, S_LOC), :] = kv_buf[
                        slot, 1, b * S_LOC:(b + 1) * S_LOC, hh * DH:(hh + 1) * DH
                    ]

        r_own_x = start_flow(0, me, 1, partners[0])
        r_own_y = start_flow(1, me, 1, partners[1])
        r_own_z = start_flow(2, me, 1, partners[2])

        wq = wq_ref[...].astype(jnp.bfloat16)
        q = jnp.dot(xf, wq, preferred_element_type=jnp.float32)
        q_rot = q * cosf + jnp.dot(q, r_mat, preferred_element_type=jnp.float32) * sinf
        qb = q_rot.astype(jnp.bfloat16)
        scores_for(me)

        r_own_x.wait_recv()
        r_fwd_y = start_flow(3, me ^ 1, 1, partners[1])
        r_fwd_z = start_flow(4, me ^ 1, 1, partners[2])
        scores_for(me ^ 1)
        r_own_y.wait_recv()
        r_fwd3_z = start_flow(5, me ^ 3, 1, partners[2])
        scores_for(me ^ 3)
        r_fwd_y.wait_recv()
        scores_for(me ^ 2)
        r_own_z.wait_recv()
        scores_for(me ^ 4)
        r_fwd_z.wait_recv()
        r_fwd5_y = start_flow(6, me ^ 5, 1, partners[1])
        scores_for(me ^ 5)
        r_fwd3_z.wait_recv()
        scores_for(me ^ 7)
        r_fwd5_y.wait_recv()
        scores_for(me ^ 6)
        for r in (r_own_x, r_own_y, r_own_z, r_fwd_y, r_fwd_z, r_fwd3_z,
                  r_fwd5_y):
            r.wait_send()

        for b in range(B):
            ctx_heads = []
            for hh in range(HQ):
                scores = scores_buf[b, hh].astype(jnp.float32)
                w = jnp.exp(scores - m_run[b][hh])
                denom = jnp.sum(w, axis=1, keepdims=True)
                acc = jnp.dot(
                    w.astype(jnp.bfloat16), v_t[b, hh],
                    preferred_element_type=jnp.float32,
                )
                ctx_heads.append(acc / denom)
            ctx_b = jnp.concatenate(ctx_heads, axis=1).astype(jnp.bfloat16)
            out_ref[b] = jnp.dot(
                ctx_b, wo_ref[...].astype(jnp.bfloat16),
                preferred_element_type=jnp.float32,
            )

    return pl.pallas_call(
        body,
        out_shape=jax.ShapeDtypeStruct((B, S_LOC, D), jnp.float32),
        in_specs=[pl.BlockSpec(memory_space=pltpu.VMEM)] * 7,
        out_specs=pl.BlockSpec(memory_space=pltpu.VMEM),
        scratch_shapes=[
            pltpu.VMEM((N_DEV, 2, R, HD), jnp.bfloat16),
            pltpu.VMEM((B, HQ, S_LOC, N_DEV * S_LOC), jnp.bfloat16),
            pltpu.VMEM((B, HQ, N_DEV * S_LOC, DH), jnp.bfloat16),
            pltpu.SemaphoreType.DMA((7,)),
            pltpu.SemaphoreType.DMA((7,)),
        ],
        compiler_params=pltpu.CompilerParams(collective_id=0),
    )(x, Wq, Wk, Wv, Wo, cos_full, sin_full)


# device time: 23501 ns/iter; 1.3139x vs baseline; 1.0052x over previous
import jax
import jax.numpy as jnp
from jax import lax
from jax.experimental import pallas as pl
from jax.experimental.pallas import tpu as pltpu

N_DEV = 8
B = 2
S_LOC = 128
D = 512
HQ = 4
DH = 64
HD = HQ * DH
R = B * S_LOC


def kernel(x, Wq, Wk, Wv, Wo):
    my = lax.axis_index("i")

    pos = (my * S_LOC).astype(jnp.float32) + jnp.arange(S_LOC, dtype=jnp.float32)
    inv = 1.0 / (10000.0 ** (jnp.arange(0, DH, 2, dtype=jnp.float32) / DH))
    ang = pos[:, None] * inv[None, :]
    cos = jnp.repeat(jnp.cos(ang), 2, axis=1)
    sin = jnp.repeat(jnp.sin(ang), 2, axis=1)
    cos_full = jnp.tile(cos, (B, HQ))
    sin_full = jnp.tile(sin, (B, HQ))

    def body(x_ref, wq_ref, wk_ref, wv_ref, wo_ref, cos_ref, sin_ref,
             out_ref, kv_buf, scores_buf, v_t, send_sems, recv_sems):
        me = lax.axis_index("i")
        partners = [me ^ 1, me ^ 3, me ^ 4]

        barrier_sem = pltpu.get_barrier_semaphore()
        for nbr in partners:
            pl.semaphore_signal(
                barrier_sem, inc=1,
                device_id=(nbr,), device_id_type=pl.DeviceIdType.MESH,
            )
        pl.semaphore_wait(barrier_sem, len(partners))

        xf = jnp.concatenate([x_ref[0], x_ref[1]], axis=0)

        row = lax.broadcasted_iota(jnp.int32, (HD, HD), 0)
        col = lax.broadcasted_iota(jnp.int32, (HD, HD), 1)
        even = (col % 2) == 0
        r_mat = jnp.where(even & (row == col + 1), -1.0, 0.0) + jnp.where(
            (~even) & (row == col - 1), 1.0, 0.0
        )
        cosf = cos_ref[...]
        sinf = sin_ref[...]

        def start_flow(idx, slot, half, partner):
            rdma = pltpu.make_async_remote_copy(
                src_ref=kv_buf.at[slot, half],
                dst_ref=kv_buf.at[slot, half],
                send_sem=send_sems.at[idx],
                recv_sem=recv_sems.at[idx],
                device_id=(partner,),
                device_id_type=pl.DeviceIdType.MESH,
            )
            rdma.start()
            return rdma

        k = jnp.dot(xf, wk_ref[...], preferred_element_type=jnp.float32)
        k_rot = k * cosf + jnp.dot(k, r_mat, preferred_element_type=jnp.float32) * sinf
        kv_buf[me, 0] = k_rot.astype(jnp.bfloat16)
        rk_own_x = start_flow(0, me, 0, partners[0])
        rk_own_y = start_flow(1, me, 0, partners[1])
        rk_own_z = start_flow(2, me, 0, partners[2])

        v = jnp.dot(xf, wv_ref[...], preferred_element_type=jnp.float32)
        kv_buf[me, 1] = v.astype(jnp.bfloat16)
        rv_own_x = start_flow(7, me, 1, partners[0])
        rv_own_y = start_flow(8, me, 1, partners[1])
        rv_own_z = start_flow(9, me, 1, partners[2])

        q = jnp.dot(xf, wq_ref[...], preferred_element_type=jnp.float32)
        q_rot = q * cosf + jnp.dot(q, r_mat, preferred_element_type=jnp.float32) * sinf
        qb = q_rot.astype(jnp.bfloat16)

        m_run = [[jnp.full((S_LOC, 1), -1e30, jnp.float32) for _ in range(HQ)]
                 for _ in range(B)]

        def scores_for(slot):
            for b in range(B):
                for hh in range(HQ):
                    qh = qb[b * S_LOC:(b + 1) * S_LOC, hh * DH:(hh + 1) * DH]
                    kc = kv_buf[slot, 0, b * S_LOC:(b + 1) * S_LOC,
                                hh * DH:(hh + 1) * DH]
                    sc = lax.dot_general(
                        qh, kc, (((1,), (1,)), ((), ())),
                        preferred_element_type=jnp.float32,
                    ) * 0.125
                    scores_buf[b, hh, :, pl.ds(slot * S_LOC, S_LOC)] = (
                        sc.astype(jnp.bfloat16)
                    )
                    m_run[b][hh] = jnp.maximum(
                        m_run[b][hh], jnp.max(sc, axis=1, keepdims=True)
                    )

        def v_repack(slot):
            for b in range(B):
                for hh in range(HQ):
                    v_t[b, hh, pl.ds(slot * S_LOC, S_LOC), :] = kv_buf[
                        slot, 1, b * S_LOC:(b + 1) * S_LOC, hh * DH:(hh + 1) * DH
                    ]

        scores_for(me)
        v_repack(me)

        rk_own_x.wait_recv()
        rk_fwd_y = start_flow(3, me ^ 1, 0, partners[1])
        rk_fwd_z = start_flow(4, me ^ 1, 0, partners[2])
        scores_for(me ^ 1)
        rv_own_x.wait_recv()
        rv_fwd_y = start_flow(10, me ^ 1, 1, partners[1])
        rv_fwd_z = start_flow(11, me ^ 1, 1, partners[2])
        v_repack(me ^ 1)

        rk_own_y.wait_recv()
        rk_fwd3_z = start_flow(5, me ^ 3, 0, partners[2])
        scores_for(me ^ 3)
        rv_own_y.wait_recv()
        rv_fwd3_z = start_flow(12, me ^ 3, 1, partners[2])
        v_repack(me ^ 3)

        rk_fwd_y.wait_recv()
        scores_for(me ^ 2)
        rk_own_z.wait_recv()
        scores_for(me ^ 4)
        rk_fwd_z.wait_recv()
        rk_fwd5_y = start_flow(6, me ^ 5, 0, partners[1])
        scores_for(me ^ 5)
        rv_fwd_z.wait_recv()
        rv_fwd5_y = start_flow(13, me ^ 5, 1, partners[1])
        v_repack(me ^ 5)
        rv_fwd_y.wait_recv()
        v_repack(me ^ 2)
        rv_own_z.wait_recv()
        v_repack(me ^ 4)

        rk_fwd3_z.wait_recv()
        scores_for(me ^ 7)
        rv_fwd3_z.wait_recv()
        v_repack(me ^ 7)
        rk_fwd5_y.wait_recv()
        scores_for(me ^ 6)
        rv_fwd5_y.wait_recv()
        v_repack(me ^ 6)

        for r in (rk_own_x, rk_own_y, rk_own_z, rk_fwd_y, rk_fwd_z,
                  rk_fwd3_z, rk_fwd5_y, rv_own_x, rv_own_y, rv_own_z,
                  rv_fwd_y, rv_fwd_z, rv_fwd3_z, rv_fwd5_y):
            r.wait_send()

        for b in range(B):
            ctx_heads = []
            for hh in range(HQ):
                scores = scores_buf[b, hh].astype(jnp.float32)
                w = jnp.exp(scores - m_run[b][hh])
                denom = jnp.sum(w, axis=1, keepdims=True)
                acc = jnp.dot(
                    w.astype(jnp.bfloat16), v_t[b, hh],
                    preferred_element_type=jnp.float32,
                )
                ctx_heads.append(acc / denom)
            ctx_b = jnp.concatenate(ctx_heads, axis=1).astype(jnp.bfloat16)
            out_ref[b] = jnp.dot(
                ctx_b, wo_ref[...], preferred_element_type=jnp.float32,
            )

    return pl.pallas_call(
        body,
        out_shape=jax.ShapeDtypeStruct((B, S_LOC, D), jnp.float32),
        in_specs=[pl.BlockSpec(memory_space=pltpu.VMEM)] * 7,
        out_specs=pl.BlockSpec(memory_space=pltpu.VMEM),
        scratch_shapes=[
            pltpu.VMEM((N_DEV, 2, R, HD), jnp.bfloat16),
            pltpu.VMEM((B, HQ, S_LOC, N_DEV * S_LOC), jnp.bfloat16),
            pltpu.VMEM((B, HQ, N_DEV * S_LOC, DH), jnp.bfloat16),
            pltpu.SemaphoreType.DMA((14,)),
            pltpu.SemaphoreType.DMA((14,)),
        ],
        compiler_params=pltpu.CompilerParams(collective_id=0),
    )(
        x.astype(jnp.bfloat16),
        Wq.astype(jnp.bfloat16),
        Wk.astype(jnp.bfloat16),
        Wv.astype(jnp.bfloat16),
        Wo.astype(jnp.bfloat16),
        cos_full,
        sin_full,
    )
